# Initial kernel scaffold; baseline (speedup 1.0000x reference)
import jax
import jax.numpy as jnp
from jax.experimental import pallas as pl


def kernel(x, edge_index, Wl0, bl0, Wr0, Wl1, bl1, Wr1):
    raise NotImplementedError("write your pallas kernel here")


# speedup vs baseline: 3.0883x; 3.0883x over previous
import functools

import jax
import jax.numpy as jnp
from jax import lax
from jax.experimental import pallas as pl
from jax.experimental.pallas import tpu as pltpu
from jax.experimental.pallas import tpu_sc as plsc

N = 10000
E = 320000
D = 128

NC = 2
NS = 16
NW = NC * NS
EPW = E // NW
K = 128
EPWP = 10240
NCHUNK = EPWP // K
NP = 10240
RPT = NP // NS


def _make_sc_scatter(with_counts):
    mesh = plsc.VectorSubcoreMesh(core_axis_name="c", subcore_axis_name="s")

    out_type = [jax.ShapeDtypeStruct((NC, NP, D), jnp.float32)]
    scratch = [
        pltpu.VMEM((NCHUNK, K), jnp.int32),
        pltpu.VMEM((NCHUNK, K), jnp.int32),
        pltpu.VMEM((K, D), jnp.float32),
        pltpu.VMEM_SHARED((NP, D), jnp.float32),
        pltpu.SemaphoreType.DMA,
    ]
    if with_counts:
        out_type.append(jax.ShapeDtypeStruct((NW, NP), jnp.int32))
        scratch.append(pltpu.VMEM((16 * 512,), jnp.int32))

    @functools.partial(
        pl.kernel, mesh=mesh, out_type=out_type, scratch_types=scratch,
        compiler_params=pltpu.CompilerParams(needs_layout_passes=False))
    def sc_kernel(table_hbm, src_hbm, dst_hbm, zrows_hbm, zhist_hbm,
                  out_hbm, *rest):
        if with_counts:
            cnt_hbm, src_v, dst_v, rows_v, acc, sem, hist = rest
        else:
            (src_v, dst_v, rows_v, acc, sem) = rest
        c = lax.axis_index("c")
        s = lax.axis_index("s")
        wid = c * NS + s
        pltpu.sync_copy(zrows_hbm, acc.at[pl.ds(s * RPT, RPT)])
        pltpu.sync_copy(src_hbm.at[wid], src_v)
        pltpu.sync_copy(dst_hbm.at[wid], dst_v)
        plsc.subcore_barrier()

        def body(i, carry):
            pltpu.async_copy(table_hbm.at[src_v.at[i]], rows_v, sem).wait()
            pltpu.sync_copy(rows_v, acc.at[dst_v.at[i]], add=True)
            return carry

        lax.fori_loop(0, NCHUNK, body, 0)

        if with_counts:
            lane = lax.iota(jnp.int32, 16)
            ones16 = jnp.ones((16,), jnp.int32)
            SPAN = 512

            for h in range(NP // SPAN):
                lo = h * SPAN
                pltpu.sync_copy(zhist_hbm, hist)

                def cbody(i, carry):
                    for b in range(K // 16):
                        d = dst_v[i, pl.ds(b * 16, 16)]
                        m = (d >= lo) & (d < lo + SPAN)
                        dloc = jnp.where(m, d - lo, 0) + lane * SPAN
                        plsc.addupdate_scatter(hist, [dloc], ones16, mask=m)
                    return carry

                lax.fori_loop(0, NCHUNK, cbody, 0)

                def rbody(cc, carry):
                    accv = hist[pl.ds(cc * 16, 16)]
                    for j in range(1, 16):
                        accv = accv + hist[pl.ds(j * SPAN + cc * 16, 16)]
                    hist[pl.ds(cc * 16, 16)] = accv
                    return carry

                lax.fori_loop(0, SPAN // 16, rbody, 0)
                pltpu.sync_copy(hist.at[pl.ds(0, SPAN)],
                                cnt_hbm.at[wid, pl.ds(lo, SPAN)])

        plsc.subcore_barrier()
        pltpu.sync_copy(acc.at[pl.ds(s * RPT, RPT)],
                        out_hbm.at[c, pl.ds(s * RPT, RPT)])

    return sc_kernel


_sc_scatter_l0 = _make_sc_scatter(True)
_sc_scatter_l1 = _make_sc_scatter(False)


BN = 512


def _tc_layer0(p0, p1, cnt_parts, x, wl, bl, wr):
    def body(p0_ref, p1_ref, cp_ref, x_ref, wl_ref, bl_ref, wr_ref,
             h_ref, cnt_ref):
        sums = p0_ref[:, :] + p1_ref[:, :]
        cnt = jnp.sum(cp_ref[:, :].astype(jnp.float32), axis=0)[:, None]
        cnt = jnp.maximum(cnt, 1.0)
        agg = sums / cnt
        out = (jnp.dot(agg, wl_ref[:], preferred_element_type=jnp.float32)
               + bl_ref[:]
               + jnp.dot(x_ref[:], wr_ref[:], preferred_element_type=jnp.float32))
        nrm = jnp.sqrt(jnp.sum(out * out, axis=1, keepdims=True))
        out = out / jnp.maximum(nrm, 1e-12)
        h_ref[:, :] = jnp.maximum(out, 0.0)
        cnt_ref[:, :] = cnt

    return pl.pallas_call(
        body,
        grid=(NP // BN,),
        in_specs=[
            pl.BlockSpec((BN, D), lambda i: (i, 0)),
            pl.BlockSpec((BN, D), lambda i: (i, 0)),
            pl.BlockSpec((NW, BN), lambda i: (0, i)),
            pl.BlockSpec((BN, D), lambda i: (i, 0)),
            pl.BlockSpec((D, D), lambda i: (0, 0)),
            pl.BlockSpec((1, D), lambda i: (0, 0)),
            pl.BlockSpec((D, D), lambda i: (0, 0)),
        ],
        out_specs=[
            pl.BlockSpec((BN, D), lambda i: (i, 0)),
            pl.BlockSpec((BN, 1), lambda i: (i, 0)),
        ],
        out_shape=[
            jax.ShapeDtypeStruct((NP, D), jnp.float32),
            jax.ShapeDtypeStruct((NP, 1), jnp.float32),
        ],
    )(p0, p1, cnt_parts, x, wl, bl, wr)


def _tc_layer1(q0, q1, cnt, h, wl, bl, wr):
    def body(q0_ref, q1_ref, cnt_ref, h_ref, wl_ref, bl_ref, wr_ref, o_ref):
        agg = (q0_ref[:, :] + q1_ref[:, :]) / cnt_ref[:, :]
        out = (jnp.dot(agg, wl_ref[:], preferred_element_type=jnp.float32)
               + bl_ref[:]
               + jnp.dot(h_ref[:], wr_ref[:], preferred_element_type=jnp.float32))
        nrm = jnp.sqrt(jnp.sum(out * out, axis=1, keepdims=True))
        o_ref[:, :] = out / jnp.maximum(nrm, 1e-12)

    return pl.pallas_call(
        body,
        grid=(NP // BN,),
        in_specs=[
            pl.BlockSpec((BN, D), lambda i: (i, 0)),
            pl.BlockSpec((BN, D), lambda i: (i, 0)),
            pl.BlockSpec((BN, 1), lambda i: (i, 0)),
            pl.BlockSpec((BN, D), lambda i: (i, 0)),
            pl.BlockSpec((D, D), lambda i: (0, 0)),
            pl.BlockSpec((1, D), lambda i: (0, 0)),
            pl.BlockSpec((D, D), lambda i: (0, 0)),
        ],
        out_specs=pl.BlockSpec((BN, D), lambda i: (i, 0)),
        out_shape=jax.ShapeDtypeStruct((NP, D), jnp.float32),
    )(q0, q1, cnt, h, wl, bl, wr)


def kernel(x, edge_index, Wl0, bl0, Wr0, Wl1, bl1, Wr1):
    pad = EPWP - EPW
    src2 = jnp.pad(edge_index[0].astype(jnp.int32).reshape(NW, EPW),
                   ((0, 0), (0, pad)))
    dst2 = jnp.pad(edge_index[1].astype(jnp.int32).reshape(NW, EPW),
                   ((0, 0), (0, pad)), constant_values=N)
    src3 = src2.reshape(NW, NCHUNK, K)
    dst3 = dst2.reshape(NW, NCHUNK, K)

    zrows = jnp.zeros((RPT, D), jnp.float32)
    zhist = jnp.zeros((16 * 512,), jnp.int32)

    x_pad = jnp.pad(x, ((0, NP - N), (0, 0)))

    p, cnt_parts = _sc_scatter_l0(x, src3, dst3, zrows, zhist)
    h, cnt = _tc_layer0(p[0], p[1], cnt_parts, x_pad, Wl0,
                        bl0.reshape(1, D), Wr0)

    (q,) = _sc_scatter_l1(h, src3, dst3, zrows, zhist)
    out = _tc_layer1(q[0], q[1], cnt, h, Wl1, bl1.reshape(1, D), Wr1)
    return out[:N]


# speedup vs baseline: 3.6505x; 1.1820x over previous
import functools

import jax
import jax.numpy as jnp
from jax import lax
from jax.experimental import pallas as pl
from jax.experimental.pallas import tpu as pltpu
from jax.experimental.pallas import tpu_sc as plsc

N = 10000
E = 320000
D = 128

NC = 2
NS = 16
NW = NC * NS
EPW = E // NW
K = 128
EPWP = 10240
NCHUNK = EPWP // K
NP = 10240
RPT = NP // NS

_mesh = plsc.VectorSubcoreMesh(core_axis_name="c", subcore_axis_name="s")
_sc_params = pltpu.CompilerParams(needs_layout_passes=False)


@functools.partial(
    pl.kernel, mesh=_mesh,
    out_type=jax.ShapeDtypeStruct((NC, NP, D), jnp.float32),
    scratch_types=[
        pltpu.VMEM((NCHUNK, K), jnp.int32),
        pltpu.VMEM((2, K), jnp.int32),
        pltpu.VMEM((2, K, D), jnp.float32),
        pltpu.VMEM_SHARED((NP, D), jnp.float32),
        pltpu.SemaphoreType.DMA,
        pltpu.SemaphoreType.DMA,
        pltpu.SemaphoreType.DMA,
    ],
    compiler_params=_sc_params)
def _sc_feature(table_hbm, src_hbm, dst_hbm, zrows_hbm, out_hbm,
                dst_v, srcb, rows_v, acc, isem, gsem, ssem):
    c = lax.axis_index("c")
    s = lax.axis_index("s")
    wid = c * NS + s
    pltpu.sync_copy(zrows_hbm, acc.at[pl.ds(s * RPT, RPT)])
    pltpu.sync_copy(dst_hbm.at[wid], dst_v)
    plsc.subcore_barrier()

    pltpu.sync_copy(src_hbm.at[wid, 0], srcb.at[0])
    pltpu.async_copy(table_hbm.at[srcb.at[0]], rows_v.at[0], gsem)
    pltpu.async_copy(src_hbm.at[wid, 1], srcb.at[1], isem)
    pltpu.make_async_copy(table_hbm.at[srcb.at[0]], rows_v.at[0], gsem).wait()
    pltpu.async_copy(rows_v.at[0], acc.at[dst_v.at[0]], ssem, add=True)
    pltpu.make_async_copy(src_hbm.at[wid, 1], srcb.at[1], isem).wait()
    pltpu.async_copy(table_hbm.at[srcb.at[1]], rows_v.at[1], gsem)
    pltpu.async_copy(src_hbm.at[wid, 2], srcb.at[0], isem)

    def body(i, carry):
        b = i % 2
        nb = 1 - b
        pltpu.make_async_copy(table_hbm.at[srcb.at[b]], rows_v.at[b],
                              gsem).wait()
        pltpu.make_async_copy(rows_v.at[nb], acc.at[dst_v.at[i - 1]],
                              ssem).wait()
        pltpu.async_copy(rows_v.at[b], acc.at[dst_v.at[i]], ssem, add=True)

        @pl.when(i + 1 < NCHUNK)
        def _():
            pltpu.make_async_copy(src_hbm.at[wid, i + 1], srcb.at[nb],
                                  isem).wait()
            pltpu.async_copy(table_hbm.at[srcb.at[nb]], rows_v.at[nb], gsem)

        @pl.when(i + 2 < NCHUNK)
        def _():
            pltpu.async_copy(src_hbm.at[wid, i + 2], srcb.at[b], isem)

        return carry

    lax.fori_loop(1, NCHUNK, body, 0)
    lb = (NCHUNK - 1) % 2
    pltpu.make_async_copy(rows_v.at[lb], acc.at[dst_v.at[NCHUNK - 1]],
                          ssem).wait()

    plsc.subcore_barrier()
    pltpu.sync_copy(acc.at[pl.ds(s * RPT, RPT)],
                    out_hbm.at[c, pl.ds(s * RPT, RPT)])


SPAN = 5120


@functools.partial(
    pl.kernel, mesh=_mesh,
    out_type=jax.ShapeDtypeStruct((NW, NP), jnp.int32),
    scratch_types=[
        pltpu.VMEM((NCHUNK, K), jnp.int32),
        pltpu.VMEM((16 * SPAN,), jnp.int32),
    ],
    compiler_params=_sc_params)
def _sc_counts(dst_hbm, zhist_hbm, cnt_hbm, dst_v, hist):
    c = lax.axis_index("c")
    s = lax.axis_index("s")
    wid = c * NS + s
    pltpu.sync_copy(dst_hbm.at[wid], dst_v)
    lane = lax.iota(jnp.int32, 16)
    ones16 = jnp.ones((16,), jnp.int32)

    for h in range(NP // SPAN):
        lo = h * SPAN
        pltpu.sync_copy(zhist_hbm, hist)

        def cbody(i, carry):
            for b in range(K // 16):
                d = dst_v[i, pl.ds(b * 16, 16)]
                m = (d >= lo) & (d < lo + SPAN)
                dloc = jnp.where(m, d - lo, 0) + lane * SPAN
                plsc.addupdate_scatter(hist, [dloc], ones16, mask=m)
            return carry

        lax.fori_loop(0, NCHUNK, cbody, 0)

        def rbody(cc, carry):
            accv = hist[pl.ds(cc * 16, 16)]
            for j in range(1, 16):
                accv = accv + hist[pl.ds(j * SPAN + cc * 16, 16)]
            hist[pl.ds(cc * 16, 16)] = accv
            return carry

        lax.fori_loop(0, SPAN // 16, rbody, 0)
        pltpu.sync_copy(hist.at[pl.ds(0, SPAN)],
                        cnt_hbm.at[wid, pl.ds(lo, SPAN)])


BN = 512


def _tc_layer0(p0, p1, cnt_parts, x, wl, bl, wr):
    def body(p0_ref, p1_ref, cp_ref, x_ref, wl_ref, bl_ref, wr_ref,
             h_ref, cnt_ref):
        sums = p0_ref[:, :] + p1_ref[:, :]
        cnt = jnp.sum(cp_ref[:, :].astype(jnp.float32), axis=0)[:, None]
        cnt = jnp.maximum(cnt, 1.0)
        agg = sums / cnt
        out = (jnp.dot(agg, wl_ref[:], preferred_element_type=jnp.float32)
               + bl_ref[:]
               + jnp.dot(x_ref[:], wr_ref[:], preferred_element_type=jnp.float32))
        nrm = jnp.sqrt(jnp.sum(out * out, axis=1, keepdims=True))
        out = out / jnp.maximum(nrm, 1e-12)
        h_ref[:, :] = jnp.maximum(out, 0.0)
        cnt_ref[:, :] = cnt

    return pl.pallas_call(
        body,
        grid=(NP // BN,),
        in_specs=[
            pl.BlockSpec((BN, D), lambda i: (i, 0)),
            pl.BlockSpec((BN, D), lambda i: (i, 0)),
            pl.BlockSpec((NW, BN), lambda i: (0, i)),
            pl.BlockSpec((BN, D), lambda i: (i, 0)),
            pl.BlockSpec((D, D), lambda i: (0, 0)),
            pl.BlockSpec((1, D), lambda i: (0, 0)),
            pl.BlockSpec((D, D), lambda i: (0, 0)),
        ],
        out_specs=[
            pl.BlockSpec((BN, D), lambda i: (i, 0)),
            pl.BlockSpec((BN, 1), lambda i: (i, 0)),
        ],
        out_shape=[
            jax.ShapeDtypeStruct((NP, D), jnp.float32),
            jax.ShapeDtypeStruct((NP, 1), jnp.float32),
        ],
    )(p0, p1, cnt_parts, x, wl, bl, wr)


def _tc_layer1(q0, q1, cnt, h, wl, bl, wr):
    def body(q0_ref, q1_ref, cnt_ref, h_ref, wl_ref, bl_ref, wr_ref, o_ref):
        agg = (q0_ref[:, :] + q1_ref[:, :]) / cnt_ref[:, :]
        out = (jnp.dot(agg, wl_ref[:], preferred_element_type=jnp.float32)
               + bl_ref[:]
               + jnp.dot(h_ref[:], wr_ref[:], preferred_element_type=jnp.float32))
        nrm = jnp.sqrt(jnp.sum(out * out, axis=1, keepdims=True))
        o_ref[:, :] = out / jnp.maximum(nrm, 1e-12)

    return pl.pallas_call(
        body,
        grid=(NP // BN,),
        in_specs=[
            pl.BlockSpec((BN, D), lambda i: (i, 0)),
            pl.BlockSpec((BN, D), lambda i: (i, 0)),
            pl.BlockSpec((BN, 1), lambda i: (i, 0)),
            pl.BlockSpec((BN, D), lambda i: (i, 0)),
            pl.BlockSpec((D, D), lambda i: (0, 0)),
            pl.BlockSpec((1, D), lambda i: (0, 0)),
            pl.BlockSpec((D, D), lambda i: (0, 0)),
        ],
        out_specs=pl.BlockSpec((BN, D), lambda i: (i, 0)),
        out_shape=jax.ShapeDtypeStruct((NP, D), jnp.float32),
    )(q0, q1, cnt, h, wl, bl, wr)


def kernel(x, edge_index, Wl0, bl0, Wr0, Wl1, bl1, Wr1):
    pad = EPWP - EPW
    src2 = jnp.pad(edge_index[0].astype(jnp.int32).reshape(NW, EPW),
                   ((0, 0), (0, pad)))
    dst2 = jnp.pad(edge_index[1].astype(jnp.int32).reshape(NW, EPW),
                   ((0, 0), (0, pad)), constant_values=N)
    src3 = src2.reshape(NW, NCHUNK, K)
    dst3 = dst2.reshape(NW, NCHUNK, K)

    zrows = jnp.zeros((RPT, D), jnp.float32)
    zhist = jnp.zeros((16 * SPAN,), jnp.int32)

    x_pad = jnp.pad(x, ((0, NP - N), (0, 0)))

    cnt_parts = _sc_counts(dst3, zhist)
    p = _sc_feature(x, src3, dst3, zrows)
    h, cnt = _tc_layer0(p[0], p[1], cnt_parts, x_pad, Wl0,
                        bl0.reshape(1, D), Wr0)

    q = _sc_feature(h, src3, dst3, zrows)
    out = _tc_layer1(q[0], q[1], cnt, h, Wl1, bl1.reshape(1, D), Wr1)
    return out[:N]


# speedup vs baseline: 3.7522x; 1.0278x over previous
import functools

import jax
import jax.numpy as jnp
from jax import lax
from jax.experimental import pallas as pl
from jax.experimental.pallas import tpu as pltpu
from jax.experimental.pallas import tpu_sc as plsc

N = 10000
E = 320000
D = 128

NC = 2
NS = 16
NW = NC * NS
EPW = E // NW
K = 80
EPWP = 10240
NCHUNK = EPWP // K
NB = 4
NI = 8
NP = 10240
RPT = NP // NS

_mesh = plsc.VectorSubcoreMesh(core_axis_name="c", subcore_axis_name="s")
_sc_params = pltpu.CompilerParams(needs_layout_passes=False)


@functools.partial(
    pl.kernel, mesh=_mesh,
    out_type=jax.ShapeDtypeStruct((NC, NP, D), jnp.float32),
    scratch_types=[
        pltpu.VMEM((NI, K), jnp.int32),
        pltpu.VMEM((NI, K), jnp.int32),
        pltpu.VMEM((NB, K, D), jnp.float32),
        pltpu.VMEM_SHARED((NP, D), jnp.float32),
        [pltpu.SemaphoreType.DMA] * NI,
        [pltpu.SemaphoreType.DMA] * NI,
        [pltpu.SemaphoreType.DMA] * NB,
        [pltpu.SemaphoreType.DMA] * NB,
    ],
    compiler_params=_sc_params)
def _sc_feature(table_hbm, src_hbm, dst_hbm, zrows_hbm, out_hbm,
                srcb, dstb, rows_v, acc, isems, jsems, gsems, ssems):
    c = lax.axis_index("c")
    s = lax.axis_index("s")
    wid = c * NS + s
    pltpu.sync_copy(zrows_hbm, acc.at[pl.ds(s * RPT, RPT)])
    plsc.subcore_barrier()

    def idx_start(ch, sl):
        pltpu.async_copy(src_hbm.at[wid, ch], srcb.at[sl], isems[sl])
        pltpu.async_copy(dst_hbm.at[wid, ch], dstb.at[sl], jsems[sl])

    def gather_start(ch, sl):
        pltpu.make_async_copy(src_hbm.at[wid, ch], srcb.at[sl],
                              isems[sl]).wait()
        pltpu.async_copy(table_hbm.at[srcb.at[sl]], rows_v.at[sl % NB],
                         gsems[sl % NB])

    for ch in range(NB):
        idx_start(ch, ch)
    for ch in range(2):
        gather_start(ch, ch)

    def group(g, carry):
        for k in range(NI):
            ch = g * NI + k
            b = k % NB
            pltpu.make_async_copy(table_hbm.at[srcb.at[k]], rows_v.at[b],
                                  gsems[b]).wait()
            pltpu.make_async_copy(dst_hbm.at[wid, ch], dstb.at[k],
                                  jsems[k]).wait()
            pltpu.async_copy(rows_v.at[b], acc.at[dstb.at[k]], ssems[b],
                             add=True)

            @pl.when(ch + 2 < NCHUNK)
            def _():
                b2 = (k + 2) % NB

                @pl.when(ch >= 2)
                def _():
                    pltpu.make_async_copy(
                        rows_v.at[b2], acc.at[dstb.at[(k + 6) % NI]],
                        ssems[b2]).wait()

                gather_start(ch + 2, (k + 2) % NI)

            @pl.when(ch + NB < NCHUNK)
            def _():
                idx_start(ch + NB, (k + NB) % NI)
        return carry

    lax.fori_loop(0, NCHUNK // NI, group, 0)
    for ch in (NCHUNK - 2, NCHUNK - 1):
        k = ch % NI
        pltpu.make_async_copy(rows_v.at[k % NB], acc.at[dstb.at[k]],
                              ssems[k % NB]).wait()

    plsc.subcore_barrier()
    pltpu.sync_copy(acc.at[pl.ds(s * RPT, RPT)],
                    out_hbm.at[c, pl.ds(s * RPT, RPT)])


SPAN = 5120


@functools.partial(
    pl.kernel, mesh=_mesh,
    out_type=jax.ShapeDtypeStruct((NW, NP), jnp.int32),
    scratch_types=[
        pltpu.VMEM((NCHUNK, K), jnp.int32),
        pltpu.VMEM((16 * SPAN,), jnp.int32),
    ],
    compiler_params=_sc_params)
def _sc_counts(dst_hbm, zhist_hbm, cnt_hbm, dst_v, hist):
    c = lax.axis_index("c")
    s = lax.axis_index("s")
    wid = c * NS + s
    pltpu.sync_copy(dst_hbm.at[wid], dst_v)
    lane = lax.iota(jnp.int32, 16)
    ones16 = jnp.ones((16,), jnp.int32)

    for h in range(NP // SPAN):
        lo = h * SPAN
        pltpu.sync_copy(zhist_hbm, hist)

        def cbody(i, carry):
            for b in range(K // 16):
                d = dst_v[i, pl.ds(b * 16, 16)]
                m = (d >= lo) & (d < lo + SPAN)
                dloc = jnp.where(m, d - lo, 0) + lane * SPAN
                plsc.addupdate_scatter(hist, [dloc], ones16, mask=m)
            return carry

        lax.fori_loop(0, NCHUNK, cbody, 0)

        def rbody(cc, carry):
            accv = hist[pl.ds(cc * 16, 16)]
            for j in range(1, 16):
                accv = accv + hist[pl.ds(j * SPAN + cc * 16, 16)]
            hist[pl.ds(cc * 16, 16)] = accv
            return carry

        lax.fori_loop(0, SPAN // 16, rbody, 0)
        pltpu.sync_copy(hist.at[pl.ds(0, SPAN)],
                        cnt_hbm.at[wid, pl.ds(lo, SPAN)])


BN = 512


def _tc_layer0(p0, p1, cnt_parts, x, wl, bl, wr):
    def body(p0_ref, p1_ref, cp_ref, x_ref, wl_ref, bl_ref, wr_ref,
             h_ref, cnt_ref):
        sums = p0_ref[:, :] + p1_ref[:, :]
        cnt = jnp.sum(cp_ref[:, :].astype(jnp.float32), axis=0)[:, None]
        cnt = jnp.maximum(cnt, 1.0)
        agg = sums / cnt
        out = (jnp.dot(agg, wl_ref[:], preferred_element_type=jnp.float32)
               + bl_ref[:]
               + jnp.dot(x_ref[:], wr_ref[:], preferred_element_type=jnp.float32))
        nrm = jnp.sqrt(jnp.sum(out * out, axis=1, keepdims=True))
        out = out / jnp.maximum(nrm, 1e-12)
        h_ref[:, :] = jnp.maximum(out, 0.0)
        cnt_ref[:, :] = cnt

    return pl.pallas_call(
        body,
        grid=(NP // BN,),
        in_specs=[
            pl.BlockSpec((BN, D), lambda i: (i, 0)),
            pl.BlockSpec((BN, D), lambda i: (i, 0)),
            pl.BlockSpec((NW, BN), lambda i: (0, i)),
            pl.BlockSpec((BN, D), lambda i: (i, 0)),
            pl.BlockSpec((D, D), lambda i: (0, 0)),
            pl.BlockSpec((1, D), lambda i: (0, 0)),
            pl.BlockSpec((D, D), lambda i: (0, 0)),
        ],
        out_specs=[
            pl.BlockSpec((BN, D), lambda i: (i, 0)),
            pl.BlockSpec((BN, 1), lambda i: (i, 0)),
        ],
        out_shape=[
            jax.ShapeDtypeStruct((NP, D), jnp.float32),
            jax.ShapeDtypeStruct((NP, 1), jnp.float32),
        ],
    )(p0, p1, cnt_parts, x, wl, bl, wr)


def _tc_layer1(q0, q1, cnt, h, wl, bl, wr):
    def body(q0_ref, q1_ref, cnt_ref, h_ref, wl_ref, bl_ref, wr_ref, o_ref):
        agg = (q0_ref[:, :] + q1_ref[:, :]) / cnt_ref[:, :]
        out = (jnp.dot(agg, wl_ref[:], preferred_element_type=jnp.float32)
               + bl_ref[:]
               + jnp.dot(h_ref[:], wr_ref[:], preferred_element_type=jnp.float32))
        nrm = jnp.sqrt(jnp.sum(out * out, axis=1, keepdims=True))
        o_ref[:, :] = out / jnp.maximum(nrm, 1e-12)

    return pl.pallas_call(
        body,
        grid=(NP // BN,),
        in_specs=[
            pl.BlockSpec((BN, D), lambda i: (i, 0)),
            pl.BlockSpec((BN, D), lambda i: (i, 0)),
            pl.BlockSpec((BN, 1), lambda i: (i, 0)),
            pl.BlockSpec((BN, D), lambda i: (i, 0)),
            pl.BlockSpec((D, D), lambda i: (0, 0)),
            pl.BlockSpec((1, D), lambda i: (0, 0)),
            pl.BlockSpec((D, D), lambda i: (0, 0)),
        ],
        out_specs=pl.BlockSpec((BN, D), lambda i: (i, 0)),
        out_shape=jax.ShapeDtypeStruct((NP, D), jnp.float32),
    )(q0, q1, cnt, h, wl, bl, wr)


def kernel(x, edge_index, Wl0, bl0, Wr0, Wl1, bl1, Wr1):
    pad = EPWP - EPW
    src2 = jnp.pad(edge_index[0].astype(jnp.int32).reshape(NW, EPW),
                   ((0, 0), (0, pad)))
    dst2 = jnp.pad(edge_index[1].astype(jnp.int32).reshape(NW, EPW),
                   ((0, 0), (0, pad)), constant_values=N)
    src3 = src2.reshape(NW, NCHUNK, K)
    dst3 = dst2.reshape(NW, NCHUNK, K)

    zrows = jnp.zeros((RPT, D), jnp.float32)
    zhist = jnp.zeros((16 * SPAN,), jnp.int32)

    x_pad = jnp.pad(x, ((0, NP - N), (0, 0)))

    cnt_parts = _sc_counts(dst3, zhist)
    p = _sc_feature(x, src3, dst3, zrows)
    h, cnt = _tc_layer0(p[0], p[1], cnt_parts, x_pad, Wl0,
                        bl0.reshape(1, D), Wr0)

    q = _sc_feature(h, src3, dst3, zrows)
    out = _tc_layer1(q[0], q[1], cnt, h, Wl1, bl1.reshape(1, D), Wr1)
    return out[:N]
